# baseline (speedup 1.0000x reference)
import functools

import jax
import jax.numpy as jnp
from jax import lax
from jax.experimental import pallas as pl
from jax.experimental.pallas import tpu as pltpu
from jax.experimental.pallas import tpu_sc as plsc

N = 10000
E = 320000
M = 128
NCRYS = 1024

_NC = 2
_NS = 16
_NW = _NC * _NS
_GCH = 128


def _leaky(x):
    return jnp.where(x >= 0, x, 0.2 * x)


_GGRP = 3
_GROWS = _GGRP * _GCH


def _sc_gather_body(table_hbm, idx_hbm, out_hbm, idx_v, ring_v, sem_g, sem_w):
    wid = lax.axis_index("s") * _NC + lax.axis_index("c")
    nch = idx_hbm.shape[1]
    ngrp = nch // _GGRP
    base = wid * (nch * _GCH)

    pltpu.sync_copy(idx_hbm.at[wid], idx_v)

    def fire(g):
        slot = (g % 2) * _GROWS
        for j in range(_GGRP):
            pltpu.async_copy(
                table_hbm.at[idx_v.at[g * _GGRP + j]],
                ring_v.at[pl.ds(slot + j * _GCH, _GCH)], sem_g)

    def drain_g():
        pltpu.make_async_copy(
            out_hbm.at[pl.ds(0, _GROWS)],
            ring_v.at[pl.ds(0, _GROWS)], sem_g).wait()

    def drain_w():
        pltpu.make_async_copy(
            ring_v.at[pl.ds(0, _GROWS)],
            out_hbm.at[pl.ds(0, _GROWS)], sem_w).wait()

    fire(0)

    def body(g, _):
        @pl.when(g >= 1)
        def _wait_wb():
            drain_w()

        @pl.when(g + 1 < ngrp)
        def _next():
            fire(g + 1)

        drain_g()
        pltpu.async_copy(
            ring_v.at[pl.ds((g % 2) * _GROWS, _GROWS)],
            out_hbm.at[pl.ds(base + g * _GROWS, _GROWS)], sem_w)
        return 0

    lax.fori_loop(0, ngrp, body, 0)
    drain_w()


def _sc_gather(table, idx_3d):
    nch = idx_3d.shape[1]
    ep = _NW * nch * _GCH
    kfn = pl.kernel(
        _sc_gather_body,
        out_type=jax.ShapeDtypeStruct((ep, M), jnp.float32),
        mesh=plsc.VectorSubcoreMesh(core_axis_name="c", subcore_axis_name="s"),
        scratch_types=[
            pltpu.VMEM((nch, _GCH), jnp.int32),
            pltpu.VMEM((2 * _GROWS, M), jnp.float32),
            pltpu.SemaphoreType.DMA,
            pltpu.SemaphoreType.DMA,
        ],
    )
    return kfn(table, idx_3d)


def _sc_scatter_body(vals_hbm, idx_hbm, zeros_hbm, out_hbm,
                     idx_v, rows_v, idx_t, rows_t, accum, sem):
    cid = lax.axis_index("c")
    sid = lax.axis_index("s")
    wid = sid * _NC + cid
    per_w = vals_hbm.shape[0] // _NW
    nfull = per_w // _GCH
    tail = per_w - nfull * _GCH
    base = wid * per_w

    @pl.when(sid == 0)
    def _init():
        pltpu.sync_copy(zeros_hbm, accum)

    plsc.subcore_barrier()

    def chunk(c, _):
        off = base + c * _GCH
        pltpu.sync_copy(idx_hbm.at[pl.ds(off, _GCH)], idx_v)
        pltpu.sync_copy(vals_hbm.at[pl.ds(off, _GCH)], rows_v)
        pltpu.sync_copy(rows_v, accum.at[idx_v], add=True)
        return 0

    lax.fori_loop(0, nfull, chunk, 0)

    if tail:
        toff = base + nfull * _GCH
        pltpu.sync_copy(idx_hbm.at[pl.ds(toff, tail)], idx_t)
        pltpu.sync_copy(vals_hbm.at[pl.ds(toff, tail)], rows_t)
        pltpu.sync_copy(rows_t, accum.at[idx_t], add=True)

    plsc.subcore_barrier()

    stripe = (accum.shape[0] // _NS) // 8 * 8
    last = accum.shape[0] - stripe * (_NS - 1)

    @pl.when(sid < _NS - 1)
    def _dump_main():
        pltpu.sync_copy(
            accum.at[pl.ds(sid * stripe, stripe)],
            out_hbm.at[pl.ds(cid * accum.shape[0] + sid * stripe, stripe)])

    @pl.when(sid == _NS - 1)
    def _dump_last():
        pltpu.sync_copy(
            accum.at[pl.ds(stripe * (_NS - 1), last)],
            out_hbm.at[pl.ds(cid * accum.shape[0] + stripe * (_NS - 1), last)])


def _sc_scatter(vals, idx, zeros_nm):
    per_w = vals.shape[0] // _NW
    tail = per_w - (per_w // _GCH) * _GCH
    kfn = pl.kernel(
        _sc_scatter_body,
        out_type=jax.ShapeDtypeStruct((2 * N, M), jnp.float32),
        mesh=plsc.VectorSubcoreMesh(core_axis_name="c", subcore_axis_name="s"),
        scratch_types=[
            pltpu.VMEM((_GCH,), jnp.int32),
            pltpu.VMEM((_GCH, M), jnp.float32),
            pltpu.VMEM((max(tail, 8),), jnp.int32),
            pltpu.VMEM((max(tail, 8), M), jnp.float32),
            pltpu.VMEM_SHARED((N, M), jnp.float32),
            pltpu.SemaphoreType.DMA,
        ],
    )
    return kfn(vals, idx, zeros_nm)


def _embed_body(x_ref, w_ref, b_ref, o_ref):
    o_ref[...] = (
        jnp.dot(x_ref[...], w_ref[...], preferred_element_type=jnp.float32, precision=lax.Precision.HIGHEST)
        + b_ref[...])


def _embed(x, w_t, b_row, block_rows):
    n, k = x.shape
    m = w_t.shape[1]
    grid = n // block_rows
    return pl.pallas_call(
        _embed_body,
        grid=(grid,),
        in_specs=[
            pl.BlockSpec((block_rows, k), lambda i: (i, 0)),
            pl.BlockSpec((k, m), lambda i: (0, 0)),
            pl.BlockSpec((1, m), lambda i: (0, 0)),
        ],
        out_specs=pl.BlockSpec((block_rows, m), lambda i: (i, 0)),
        out_shape=jax.ShapeDtypeStruct((n, m), jnp.float32),
    )(x, w_t, b_row)


def _edge_body(g1, g2, nbr, w1a, w1b, w1c, b1, w2, b2, w3, b3, ek_o, nbr_o):
    t = jnp.dot(g1[...], w1a[...], preferred_element_type=jnp.float32, precision=lax.Precision.HIGHEST)
    t += jnp.dot(g2[...], w1b[...], preferred_element_type=jnp.float32, precision=lax.Precision.HIGHEST)
    t += jnp.dot(nbr[...], w1c[...], preferred_element_type=jnp.float32, precision=lax.Precision.HIGHEST)
    h = _leaky(t + b1[...])
    h = _leaky(jnp.dot(h, w2[...], preferred_element_type=jnp.float32, precision=lax.Precision.HIGHEST)
               + b2[...])
    ek = jnp.dot(h, w3[...], preferred_element_type=jnp.float32, precision=lax.Precision.HIGHEST) + b3[...]
    ek_o[...] = ek
    nbr_o[...] = nbr[...] + ek


def _edge_mlp(g1, g2, nbr, w1a, w1b, w1c, b1, w2, b2, w3, b3, block_rows):
    grid = E // block_rows
    row = lambda i: (i, 0)
    fix = lambda i: (0, 0)
    return pl.pallas_call(
        _edge_body,
        grid=(grid,),
        in_specs=[
            pl.BlockSpec((block_rows, M), row),
            pl.BlockSpec((block_rows, M), row),
            pl.BlockSpec((block_rows, M), row),
            pl.BlockSpec((M, M), fix), pl.BlockSpec((M, M), fix),
            pl.BlockSpec((M, M), fix), pl.BlockSpec((1, M), fix),
            pl.BlockSpec((M, M), fix), pl.BlockSpec((1, M), fix),
            pl.BlockSpec((M, M), fix), pl.BlockSpec((1, M), fix),
        ],
        out_specs=[
            pl.BlockSpec((block_rows, M), row),
            pl.BlockSpec((block_rows, M), row),
        ],
        out_shape=[
            jax.ShapeDtypeStruct((E, M), jnp.float32),
            jax.ShapeDtypeStruct((E, M), jnp.float32),
        ],
    )(g1, g2, nbr, w1a, w1b, w1c, b1, w2, b2, w3, b3)


def _node_body(atom, p0, p1, nn, wa, wr, b1, w2, b2, w3, b3, g, bb, out):
    rho = (p0[...] + p1[...]) / nn[...]
    t = jnp.dot(atom[...], wa[...], preferred_element_type=jnp.float32, precision=lax.Precision.HIGHEST)
    t += jnp.dot(rho, wr[...], preferred_element_type=jnp.float32, precision=lax.Precision.HIGHEST)
    h = _leaky(t + b1[...])
    h = _leaky(jnp.dot(h, w2[...], preferred_element_type=jnp.float32, precision=lax.Precision.HIGHEST)
               + b2[...])
    vi = jnp.dot(h, w3[...], preferred_element_type=jnp.float32, precision=lax.Precision.HIGHEST) + b3[...]
    mu = jnp.mean(vi, axis=0, keepdims=True)
    var = jnp.mean((vi - mu) ** 2, axis=0, keepdims=True)
    vi = (vi - mu) / jnp.sqrt(var + 1e-5) * g[...] + bb[...]
    out[...] = atom[...] + vi


def _node_mlp(atom, p0, p1, nn_col, wa, wr, b1, w2, b2, w3, b3, g_row, b_row):
    return pl.pallas_call(
        _node_body,
        out_shape=jax.ShapeDtypeStruct((N, M), jnp.float32),
    )(atom, p0, p1, nn_col, wa, wr, b1, w2, b2, w3, b3, g_row, b_row)


def _pool_body(vi, p0, p1, nn, cidx, ua, ue, ub1, uw2, ub2,
               fcw, fcb, fc1w, fc1b, ow, ob, out,
               gfa, gfb, cnt):
    i = pl.program_id(0)
    nblk = pl.num_programs(0)
    rows = vi.shape[0]

    @pl.when(i == 0)
    def _zero():
        gfa[...] = jnp.zeros_like(gfa)
        gfb[...] = jnp.zeros_like(gfb)
        cnt[...] = jnp.zeros_like(cnt)

    eks = (p0[...] + p1[...]) / nn[...]
    iota = lax.broadcasted_iota(jnp.int32, (rows, NCRYS), 1)
    onehot = (iota == cidx[...]).astype(jnp.float32)
    dn = (((0,), (0,)), ((), ()))
    gfa[...] += lax.dot_general(onehot, vi[...], dn,
                                preferred_element_type=jnp.float32, precision=lax.Precision.HIGHEST)
    gfb[...] += lax.dot_general(onehot, eks, dn,
                                preferred_element_type=jnp.float32, precision=lax.Precision.HIGHEST)
    cnt[...] += lax.dot_general(onehot, jnp.ones((rows, M), jnp.float32), dn,
                                preferred_element_type=jnp.float32, precision=lax.Precision.HIGHEST)

    @pl.when(i == nblk - 1)
    def _head():
        pa = gfa[...] / cnt[...]
        pb = gfb[...] / cnt[...]
        z = jnp.dot(pa, ua[...], preferred_element_type=jnp.float32, precision=lax.Precision.HIGHEST)
        z += jnp.dot(pb, ue[...], preferred_element_type=jnp.float32, precision=lax.Precision.HIGHEST)
        z = _leaky(z + ub1[...])
        z = jnp.tanh(jnp.dot(z, uw2[...], preferred_element_type=jnp.float32, precision=lax.Precision.HIGHEST)
                     + ub2[...])
        c = _leaky(jnp.dot(z, fcw[...], preferred_element_type=jnp.float32, precision=lax.Precision.HIGHEST)
                   + fcb[...])
        c = _leaky(jnp.dot(c, fc1w[...], preferred_element_type=jnp.float32, precision=lax.Precision.HIGHEST)
                   + fc1b[...])
        out[...] = (jnp.dot(c, ow[...], preferred_element_type=jnp.float32, precision=lax.Precision.HIGHEST)
                    + ob[...])


def _pool_head(vi, p0, p1, nn_col, cidx_col, ua, ue, ub1, uw2, ub2,
               fcw, fcb, fc1w, fc1b, ow, ob, block_rows):
    grid = N // block_rows
    row = lambda i: (i, 0)
    fix = lambda i: (0, 0)
    return pl.pallas_call(
        _pool_body,
        grid=(grid,),
        in_specs=[
            pl.BlockSpec((block_rows, M), row),
            pl.BlockSpec((block_rows, M), row),
            pl.BlockSpec((block_rows, M), row),
            pl.BlockSpec((block_rows, 1), row),
            pl.BlockSpec((block_rows, 1), row),
            pl.BlockSpec((M, M), fix), pl.BlockSpec((M, M), fix),
            pl.BlockSpec((1, M), fix),
            pl.BlockSpec((M, M), fix), pl.BlockSpec((1, M), fix),
            pl.BlockSpec((M, M), fix), pl.BlockSpec((1, M), fix),
            pl.BlockSpec((M, M), fix), pl.BlockSpec((1, M), fix),
            pl.BlockSpec((M, 1), fix), pl.BlockSpec((1, 1), fix),
        ],
        out_specs=pl.BlockSpec((NCRYS, 1), fix),
        out_shape=jax.ShapeDtypeStruct((NCRYS, 1), jnp.float32),
        scratch_shapes=[
            pltpu.VMEM((NCRYS, M), jnp.float32),
            pltpu.VMEM((NCRYS, M), jnp.float32),
            pltpu.VMEM((NCRYS, M), jnp.float32),
        ],
    )(vi, p0, p1, nn_col, cidx_col, ua, ue, ub1, uw2, ub2,
      fcw, fcb, fc1w, fc1b, ow, ob)


def kernel(atom_fea, nbr_fea, nbr_fea_idx1, nbr_fea_idx2, num_nbrs,
           crystal_atom_idx, params):
    p = params
    rowb = lambda b: b.reshape(1, -1)

    idx_all = jnp.concatenate([nbr_fea_idx1, nbr_fea_idx2])
    stride = _NW * _GCH * _GGRP
    ep = ((2 * E + stride - 1) // stride) * stride
    idx_3d = jnp.pad(idx_all, (0, ep - 2 * E)).reshape(_NW, -1, _GCH)

    zeros_nm = jnp.zeros((N, M), jnp.float32)
    nn_col = num_nbrs.reshape(N, 1)
    cidx_col = crystal_atom_idx.reshape(N, 1)

    atom = _embed(atom_fea, p["node_W"].T, rowb(p["node_b"]), 2000)
    nbr = _embed(nbr_fea, p["edge_W"].T, rowb(p["edge_b"]), 2000)

    eks_parts = None
    nconv = len(p["convs"])
    for li, c in enumerate(p["convs"]):
        gath = _sc_gather(atom, idx_3d)
        e_w1t = c["eW1"].T
        ek, nbr = _edge_mlp(
            gath[:E], gath[E:2 * E], nbr,
            e_w1t[:M], e_w1t[M:2 * M], e_w1t[2 * M:], rowb(c["eb1"]),
            c["eW2"].T, rowb(c["eb2"]), c["eW3"].T, rowb(c["eb3"]), 2000)
        rho_parts = _sc_scatter(ek, nbr_fea_idx1, zeros_nm)
        v_w1t = c["vW1"].T
        atom = _node_mlp(
            atom, rho_parts[:N], rho_parts[N:], nn_col,
            v_w1t[:M], v_w1t[M:], rowb(c["vb1"]),
            c["vW2"].T, rowb(c["vb2"]), c["vW3"].T, rowb(c["vb3"]),
            rowb(c["bn_g"]), rowb(c["bn_b"]))
        if li == nconv - 1:
            eks_parts = _sc_scatter(nbr, nbr_fea_idx1, zeros_nm)

    u_w1t = p["uW1"].T
    return _pool_head(
        atom, eks_parts[:N], eks_parts[N:], nn_col, cidx_col,
        u_w1t[:M], u_w1t[M:], rowb(p["ub1"]),
        p["uW2"].T, rowb(p["ub2"]),
        p["fcW"].T, rowb(p["fcb"]),
        p["fc1W"].T, rowb(p["fc1b"]),
        p["outW"].T, rowb(p["outb"]), 2000)


# speedup vs baseline: 1.2382x; 1.2382x over previous
import functools

import jax
import jax.numpy as jnp
from jax import lax
from jax.experimental import pallas as pl
from jax.experimental.pallas import tpu as pltpu
from jax.experimental.pallas import tpu_sc as plsc

N = 10000
E = 320000
M = 128
NCRYS = 1024

_NC = 2
_NS = 16
_NW = _NC * _NS
_GCH = 128


def _leaky(x):
    return jnp.where(x >= 0, x, 0.2 * x)


def _sc_gather_body(table_hbm, idx_hbm, out_hbm, idx_v, rows_v, sem_g, sem_w):
    wid = lax.axis_index("s") * _NC + lax.axis_index("c")
    nch = idx_hbm.shape[1]
    base = wid * (nch * _GCH)

    pltpu.sync_copy(idx_hbm.at[wid], idx_v)

    def drain_w():
        pltpu.make_async_copy(
            rows_v.at[pl.ds(0, _GCH)],
            out_hbm.at[pl.ds(0, _GCH)], sem_w).wait()

    def body(c, _):
        b = (c % 2) * _GCH

        @pl.when(c >= 2)
        def _wait_wb():
            drain_w()

        pltpu.async_copy(
            table_hbm.at[idx_v.at[c]],
            rows_v.at[pl.ds(b, _GCH)], sem_g).wait()
        pltpu.async_copy(
            rows_v.at[pl.ds(b, _GCH)],
            out_hbm.at[pl.ds(base + c * _GCH, _GCH)], sem_w)
        return 0

    lax.fori_loop(0, nch, body, 0)
    drain_w()
    drain_w()


def _sc_gather(table, idx_3d):
    nch = idx_3d.shape[1]
    ep = _NW * nch * _GCH
    kfn = pl.kernel(
        _sc_gather_body,
        out_type=jax.ShapeDtypeStruct((ep, M), jnp.float32),
        mesh=plsc.VectorSubcoreMesh(core_axis_name="c", subcore_axis_name="s"),
        scratch_types=[
            pltpu.VMEM((nch, _GCH), jnp.int32),
            pltpu.VMEM((2 * _GCH, M), jnp.float32),
            pltpu.SemaphoreType.DMA,
            pltpu.SemaphoreType.DMA,
        ],
    )
    return kfn(table, idx_3d)


def _sc_scatter_body(vals_hbm, idx_hbm, zeros_hbm, out_hbm,
                     idx_v, rows_v, idx_t, rows_t, accum, sem_l):
    cid = lax.axis_index("c")
    sid = lax.axis_index("s")
    wid = sid * _NC + cid
    per_w = vals_hbm.shape[0] // _NW
    nfull = per_w // _GCH
    tail = per_w - nfull * _GCH
    base = wid * per_w

    zstripe = (accum.shape[0] // _NS) // 8 * 8
    zlast = accum.shape[0] - zstripe * (_NS - 1)

    @pl.when(sid < _NS - 1)
    def _zero_main():
        pltpu.sync_copy(zeros_hbm.at[pl.ds(sid * zstripe, zstripe)],
                        accum.at[pl.ds(sid * zstripe, zstripe)])

    @pl.when(sid == _NS - 1)
    def _zero_last():
        pltpu.sync_copy(zeros_hbm.at[pl.ds(zstripe * (_NS - 1), zlast)],
                        accum.at[pl.ds(zstripe * (_NS - 1), zlast)])

    plsc.subcore_barrier()

    def fire(c):
        off = base + c * _GCH
        b = c % 2
        pltpu.async_copy(idx_hbm.at[pl.ds(off, _GCH)], idx_v.at[b], sem_l)
        pltpu.async_copy(vals_hbm.at[pl.ds(off, _GCH)],
                         rows_v.at[pl.ds(b * _GCH, _GCH)], sem_l)

    def drain_l():
        pltpu.make_async_copy(idx_hbm.at[pl.ds(0, _GCH)],
                              idx_v.at[0], sem_l).wait()
        pltpu.make_async_copy(vals_hbm.at[pl.ds(0, _GCH)],
                              rows_v.at[pl.ds(0, _GCH)], sem_l).wait()

    fire(0)

    def chunk(c, _):
        b = c % 2

        @pl.when(c + 1 < nfull)
        def _next():
            fire(c + 1)

        drain_l()
        pltpu.sync_copy(rows_v.at[pl.ds(b * _GCH, _GCH)],
                        accum.at[idx_v.at[b]], add=True)
        return 0

    lax.fori_loop(0, nfull, chunk, 0)

    if tail:
        toff = base + nfull * _GCH
        pltpu.sync_copy(idx_hbm.at[pl.ds(toff, tail)], idx_t)
        pltpu.sync_copy(vals_hbm.at[pl.ds(toff, tail)], rows_t)
        pltpu.sync_copy(rows_t, accum.at[idx_t], add=True)

    plsc.subcore_barrier()

    stripe = (accum.shape[0] // _NS) // 8 * 8
    last = accum.shape[0] - stripe * (_NS - 1)

    @pl.when(sid < _NS - 1)
    def _dump_main():
        pltpu.sync_copy(
            accum.at[pl.ds(sid * stripe, stripe)],
            out_hbm.at[pl.ds(cid * accum.shape[0] + sid * stripe, stripe)])

    @pl.when(sid == _NS - 1)
    def _dump_last():
        pltpu.sync_copy(
            accum.at[pl.ds(stripe * (_NS - 1), last)],
            out_hbm.at[pl.ds(cid * accum.shape[0] + stripe * (_NS - 1), last)])


def _sc_scatter(vals, idx, zeros_nm):
    per_w = vals.shape[0] // _NW
    tail = per_w - (per_w // _GCH) * _GCH
    kfn = pl.kernel(
        _sc_scatter_body,
        out_type=jax.ShapeDtypeStruct((2 * N, M), jnp.float32),
        mesh=plsc.VectorSubcoreMesh(core_axis_name="c", subcore_axis_name="s"),
        scratch_types=[
            pltpu.VMEM((2, _GCH), jnp.int32),
            pltpu.VMEM((2 * _GCH, M), jnp.float32),
            pltpu.VMEM((max(tail, 8),), jnp.int32),
            pltpu.VMEM((max(tail, 8), M), jnp.float32),
            pltpu.VMEM_SHARED((N, M), jnp.float32),
            pltpu.SemaphoreType.DMA,
        ],
    )
    return kfn(vals, idx, zeros_nm)


def _embed_body(x_ref, w_ref, b_ref, o_ref):
    o_ref[...] = (
        jnp.dot(x_ref[...], w_ref[...], preferred_element_type=jnp.float32, precision=lax.Precision.HIGHEST)
        + b_ref[...])


def _embed(x, w_t, b_row, block_rows):
    n, k = x.shape
    m = w_t.shape[1]
    grid = n // block_rows
    return pl.pallas_call(
        _embed_body,
        grid=(grid,),
        in_specs=[
            pl.BlockSpec((block_rows, k), lambda i: (i, 0)),
            pl.BlockSpec((k, m), lambda i: (0, 0)),
            pl.BlockSpec((1, m), lambda i: (0, 0)),
        ],
        out_specs=pl.BlockSpec((block_rows, m), lambda i: (i, 0)),
        out_shape=jax.ShapeDtypeStruct((n, m), jnp.float32),
    )(x, w_t, b_row)


def _edge_body(g1, g2, nbr, w1a, w1b, w1c, b1, w2, b2, w3, b3, ek_o, nbr_o):
    t = jnp.dot(g1[...], w1a[...], preferred_element_type=jnp.float32, precision=lax.Precision.HIGHEST)
    t += jnp.dot(g2[...], w1b[...], preferred_element_type=jnp.float32, precision=lax.Precision.HIGHEST)
    t += jnp.dot(nbr[...], w1c[...], preferred_element_type=jnp.float32, precision=lax.Precision.HIGHEST)
    h = _leaky(t + b1[...])
    h = _leaky(jnp.dot(h, w2[...], preferred_element_type=jnp.float32, precision=lax.Precision.HIGHEST)
               + b2[...])
    ek = jnp.dot(h, w3[...], preferred_element_type=jnp.float32, precision=lax.Precision.HIGHEST) + b3[...]
    ek_o[...] = ek
    nbr_o[...] = nbr[...] + ek


def _edge_mlp(g1, g2, nbr, w1a, w1b, w1c, b1, w2, b2, w3, b3, block_rows):
    grid = E // block_rows
    row = lambda i: (i, 0)
    fix = lambda i: (0, 0)
    return pl.pallas_call(
        _edge_body,
        grid=(grid,),
        in_specs=[
            pl.BlockSpec((block_rows, M), row),
            pl.BlockSpec((block_rows, M), row),
            pl.BlockSpec((block_rows, M), row),
            pl.BlockSpec((M, M), fix), pl.BlockSpec((M, M), fix),
            pl.BlockSpec((M, M), fix), pl.BlockSpec((1, M), fix),
            pl.BlockSpec((M, M), fix), pl.BlockSpec((1, M), fix),
            pl.BlockSpec((M, M), fix), pl.BlockSpec((1, M), fix),
        ],
        out_specs=[
            pl.BlockSpec((block_rows, M), row),
            pl.BlockSpec((block_rows, M), row),
        ],
        out_shape=[
            jax.ShapeDtypeStruct((E, M), jnp.float32),
            jax.ShapeDtypeStruct((E, M), jnp.float32),
        ],
    )(g1, g2, nbr, w1a, w1b, w1c, b1, w2, b2, w3, b3)


def _node_body(atom, p0, p1, nn, wa, wr, b1, w2, b2, w3, b3, g, bb, out):
    rho = (p0[...] + p1[...]) / nn[...]
    t = jnp.dot(atom[...], wa[...], preferred_element_type=jnp.float32, precision=lax.Precision.HIGHEST)
    t += jnp.dot(rho, wr[...], preferred_element_type=jnp.float32, precision=lax.Precision.HIGHEST)
    h = _leaky(t + b1[...])
    h = _leaky(jnp.dot(h, w2[...], preferred_element_type=jnp.float32, precision=lax.Precision.HIGHEST)
               + b2[...])
    vi = jnp.dot(h, w3[...], preferred_element_type=jnp.float32, precision=lax.Precision.HIGHEST) + b3[...]
    mu = jnp.mean(vi, axis=0, keepdims=True)
    var = jnp.mean((vi - mu) ** 2, axis=0, keepdims=True)
    vi = (vi - mu) / jnp.sqrt(var + 1e-5) * g[...] + bb[...]
    out[...] = atom[...] + vi


def _node_mlp(atom, p0, p1, nn_col, wa, wr, b1, w2, b2, w3, b3, g_row, b_row):
    return pl.pallas_call(
        _node_body,
        out_shape=jax.ShapeDtypeStruct((N, M), jnp.float32),
    )(atom, p0, p1, nn_col, wa, wr, b1, w2, b2, w3, b3, g_row, b_row)


def _pool_body(vi, p0, p1, nn, cidx, ua, ue, ub1, uw2, ub2,
               fcw, fcb, fc1w, fc1b, ow, ob, out,
               gfa, gfb, cnt):
    i = pl.program_id(0)
    nblk = pl.num_programs(0)
    rows = vi.shape[0]

    @pl.when(i == 0)
    def _zero():
        gfa[...] = jnp.zeros_like(gfa)
        gfb[...] = jnp.zeros_like(gfb)
        cnt[...] = jnp.zeros_like(cnt)

    eks = (p0[...] + p1[...]) / nn[...]
    iota = lax.broadcasted_iota(jnp.int32, (rows, NCRYS), 1)
    onehot = (iota == cidx[...]).astype(jnp.float32)
    dn = (((0,), (0,)), ((), ()))
    gfa[...] += lax.dot_general(onehot, vi[...], dn,
                                preferred_element_type=jnp.float32, precision=lax.Precision.HIGHEST)
    gfb[...] += lax.dot_general(onehot, eks, dn,
                                preferred_element_type=jnp.float32, precision=lax.Precision.HIGHEST)
    cnt[...] += lax.dot_general(onehot, jnp.ones((rows, M), jnp.float32), dn,
                                preferred_element_type=jnp.float32, precision=lax.Precision.HIGHEST)

    @pl.when(i == nblk - 1)
    def _head():
        pa = gfa[...] / cnt[...]
        pb = gfb[...] / cnt[...]
        z = jnp.dot(pa, ua[...], preferred_element_type=jnp.float32, precision=lax.Precision.HIGHEST)
        z += jnp.dot(pb, ue[...], preferred_element_type=jnp.float32, precision=lax.Precision.HIGHEST)
        z = _leaky(z + ub1[...])
        z = jnp.tanh(jnp.dot(z, uw2[...], preferred_element_type=jnp.float32, precision=lax.Precision.HIGHEST)
                     + ub2[...])
        c = _leaky(jnp.dot(z, fcw[...], preferred_element_type=jnp.float32, precision=lax.Precision.HIGHEST)
                   + fcb[...])
        c = _leaky(jnp.dot(c, fc1w[...], preferred_element_type=jnp.float32, precision=lax.Precision.HIGHEST)
                   + fc1b[...])
        out[...] = (jnp.dot(c, ow[...], preferred_element_type=jnp.float32, precision=lax.Precision.HIGHEST)
                    + ob[...])


def _pool_head(vi, p0, p1, nn_col, cidx_col, ua, ue, ub1, uw2, ub2,
               fcw, fcb, fc1w, fc1b, ow, ob, block_rows):
    grid = N // block_rows
    row = lambda i: (i, 0)
    fix = lambda i: (0, 0)
    return pl.pallas_call(
        _pool_body,
        grid=(grid,),
        in_specs=[
            pl.BlockSpec((block_rows, M), row),
            pl.BlockSpec((block_rows, M), row),
            pl.BlockSpec((block_rows, M), row),
            pl.BlockSpec((block_rows, 1), row),
            pl.BlockSpec((block_rows, 1), row),
            pl.BlockSpec((M, M), fix), pl.BlockSpec((M, M), fix),
            pl.BlockSpec((1, M), fix),
            pl.BlockSpec((M, M), fix), pl.BlockSpec((1, M), fix),
            pl.BlockSpec((M, M), fix), pl.BlockSpec((1, M), fix),
            pl.BlockSpec((M, M), fix), pl.BlockSpec((1, M), fix),
            pl.BlockSpec((M, 1), fix), pl.BlockSpec((1, 1), fix),
        ],
        out_specs=pl.BlockSpec((NCRYS, 1), fix),
        out_shape=jax.ShapeDtypeStruct((NCRYS, 1), jnp.float32),
        scratch_shapes=[
            pltpu.VMEM((NCRYS, M), jnp.float32),
            pltpu.VMEM((NCRYS, M), jnp.float32),
            pltpu.VMEM((NCRYS, M), jnp.float32),
        ],
    )(vi, p0, p1, nn_col, cidx_col, ua, ue, ub1, uw2, ub2,
      fcw, fcb, fc1w, fc1b, ow, ob)


def kernel(atom_fea, nbr_fea, nbr_fea_idx1, nbr_fea_idx2, num_nbrs,
           crystal_atom_idx, params):
    p = params
    rowb = lambda b: b.reshape(1, -1)

    idx_all = jnp.concatenate([nbr_fea_idx1, nbr_fea_idx2])
    stride = _NW * _GCH
    ep = ((2 * E + stride - 1) // stride) * stride
    idx_3d = jnp.pad(idx_all, (0, ep - 2 * E)).reshape(_NW, -1, _GCH)

    zeros_nm = jnp.zeros((N, M), jnp.float32)
    nn_col = num_nbrs.reshape(N, 1)
    cidx_col = crystal_atom_idx.reshape(N, 1)

    atom = _embed(atom_fea, p["node_W"].T, rowb(p["node_b"]), 2000)
    nbr = _embed(nbr_fea, p["edge_W"].T, rowb(p["edge_b"]), 2000)

    eks_parts = None
    nconv = len(p["convs"])
    for li, c in enumerate(p["convs"]):
        gath = _sc_gather(atom, idx_3d)
        e_w1t = c["eW1"].T
        ek, nbr = _edge_mlp(
            gath[:E], gath[E:2 * E], nbr,
            e_w1t[:M], e_w1t[M:2 * M], e_w1t[2 * M:], rowb(c["eb1"]),
            c["eW2"].T, rowb(c["eb2"]), c["eW3"].T, rowb(c["eb3"]), 2000)
        rho_parts = _sc_scatter(ek, nbr_fea_idx1, zeros_nm)
        v_w1t = c["vW1"].T
        atom = _node_mlp(
            atom, rho_parts[:N], rho_parts[N:], nn_col,
            v_w1t[:M], v_w1t[M:], rowb(c["vb1"]),
            c["vW2"].T, rowb(c["vb2"]), c["vW3"].T, rowb(c["vb3"]),
            rowb(c["bn_g"]), rowb(c["bn_b"]))
        if li == nconv - 1:
            eks_parts = _sc_scatter(nbr, nbr_fea_idx1, zeros_nm)

    u_w1t = p["uW1"].T
    return _pool_head(
        atom, eks_parts[:N], eks_parts[N:], nn_col, cidx_col,
        u_w1t[:M], u_w1t[M:], rowb(p["ub1"]),
        p["uW2"].T, rowb(p["ub2"]),
        p["fcW"].T, rowb(p["fcb"]),
        p["fc1W"].T, rowb(p["fc1b"]),
        p["outW"].T, rowb(p["outb"]), 2000)


# speedup vs baseline: 1.2613x; 1.0186x over previous
import functools

import jax
import jax.numpy as jnp
from jax import lax
from jax.experimental import pallas as pl
from jax.experimental.pallas import tpu as pltpu
from jax.experimental.pallas import tpu_sc as plsc

N = 10000
E = 320000
M = 128
NCRYS = 1024

_NC = 2
_NS = 16
_NW = _NC * _NS
_GCH = 128


def _leaky(x):
    return jnp.where(x >= 0, x, 0.2 * x)


def _sc_gather_body(table_hbm, idx_hbm, out_hbm, idx_v, rows_v, sem_g, sem_w):
    wid = lax.axis_index("s") * _NC + lax.axis_index("c")
    nch = idx_hbm.shape[1]
    base = wid * (nch * _GCH)

    pltpu.sync_copy(idx_hbm.at[wid], idx_v)

    def drain_w():
        pltpu.make_async_copy(
            rows_v.at[pl.ds(0, _GCH)],
            out_hbm.at[pl.ds(0, _GCH)], sem_w).wait()

    def fire_g(c):
        pltpu.async_copy(
            table_hbm.at[idx_v.at[c]],
            rows_v.at[pl.ds((c % 3) * _GCH, _GCH)], sem_g)

    fire_g(0)

    def body(c, _):
        @pl.when(c >= 2)
        def _wait_wb():
            drain_w()

        @pl.when(c + 1 < nch)
        def _next():
            fire_g(c + 1)

        pltpu.make_async_copy(
            out_hbm.at[pl.ds(0, _GCH)],
            rows_v.at[pl.ds(0, _GCH)], sem_g).wait()
        pltpu.async_copy(
            rows_v.at[pl.ds((c % 3) * _GCH, _GCH)],
            out_hbm.at[pl.ds(base + c * _GCH, _GCH)], sem_w)
        return 0

    lax.fori_loop(0, nch, body, 0)
    drain_w()
    drain_w()


def _sc_gather(table, idx_3d):
    nch = idx_3d.shape[1]
    ep = _NW * nch * _GCH
    kfn = pl.kernel(
        _sc_gather_body,
        out_type=jax.ShapeDtypeStruct((ep, M), jnp.float32),
        mesh=plsc.VectorSubcoreMesh(core_axis_name="c", subcore_axis_name="s"),
        scratch_types=[
            pltpu.VMEM((nch, _GCH), jnp.int32),
            pltpu.VMEM((3 * _GCH, M), jnp.float32),
            pltpu.SemaphoreType.DMA,
            pltpu.SemaphoreType.DMA,
        ],
    )
    return kfn(table, idx_3d)


def _sc_scatter_body(vals_hbm, idx_hbm, zeros_hbm, out_hbm,
                     idx_v, rows_v, idx_t, rows_t, accum, sem_l):
    cid = lax.axis_index("c")
    sid = lax.axis_index("s")
    wid = sid * _NC + cid
    per_w = vals_hbm.shape[0] // _NW
    nfull = per_w // _GCH
    tail = per_w - nfull * _GCH
    base = wid * per_w

    zstripe = (accum.shape[0] // _NS) // 8 * 8
    zlast = accum.shape[0] - zstripe * (_NS - 1)

    @pl.when(sid < _NS - 1)
    def _zero_main():
        pltpu.sync_copy(zeros_hbm.at[pl.ds(sid * zstripe, zstripe)],
                        accum.at[pl.ds(sid * zstripe, zstripe)])

    @pl.when(sid == _NS - 1)
    def _zero_last():
        pltpu.sync_copy(zeros_hbm.at[pl.ds(zstripe * (_NS - 1), zlast)],
                        accum.at[pl.ds(zstripe * (_NS - 1), zlast)])

    plsc.subcore_barrier()

    def fire(c):
        off = base + c * _GCH
        b = c % 2
        pltpu.async_copy(idx_hbm.at[pl.ds(off, _GCH)], idx_v.at[b], sem_l)
        pltpu.async_copy(vals_hbm.at[pl.ds(off, _GCH)],
                         rows_v.at[pl.ds(b * _GCH, _GCH)], sem_l)

    def drain_l():
        pltpu.make_async_copy(idx_hbm.at[pl.ds(0, _GCH)],
                              idx_v.at[0], sem_l).wait()
        pltpu.make_async_copy(vals_hbm.at[pl.ds(0, _GCH)],
                              rows_v.at[pl.ds(0, _GCH)], sem_l).wait()

    fire(0)

    def chunk(c, _):
        b = c % 2

        @pl.when(c + 1 < nfull)
        def _next():
            fire(c + 1)

        drain_l()
        pltpu.sync_copy(rows_v.at[pl.ds(b * _GCH, _GCH)],
                        accum.at[idx_v.at[b]], add=True)
        return 0

    lax.fori_loop(0, nfull, chunk, 0)

    if tail:
        toff = base + nfull * _GCH
        pltpu.sync_copy(idx_hbm.at[pl.ds(toff, tail)], idx_t)
        pltpu.sync_copy(vals_hbm.at[pl.ds(toff, tail)], rows_t)
        pltpu.sync_copy(rows_t, accum.at[idx_t], add=True)

    plsc.subcore_barrier()

    stripe = (accum.shape[0] // _NS) // 8 * 8
    last = accum.shape[0] - stripe * (_NS - 1)

    @pl.when(sid < _NS - 1)
    def _dump_main():
        pltpu.sync_copy(
            accum.at[pl.ds(sid * stripe, stripe)],
            out_hbm.at[pl.ds(cid * accum.shape[0] + sid * stripe, stripe)])

    @pl.when(sid == _NS - 1)
    def _dump_last():
        pltpu.sync_copy(
            accum.at[pl.ds(stripe * (_NS - 1), last)],
            out_hbm.at[pl.ds(cid * accum.shape[0] + stripe * (_NS - 1), last)])


def _sc_scatter(vals, idx, zeros_nm):
    per_w = vals.shape[0] // _NW
    tail = per_w - (per_w // _GCH) * _GCH
    kfn = pl.kernel(
        _sc_scatter_body,
        out_type=jax.ShapeDtypeStruct((2 * N, M), jnp.float32),
        mesh=plsc.VectorSubcoreMesh(core_axis_name="c", subcore_axis_name="s"),
        scratch_types=[
            pltpu.VMEM((2, _GCH), jnp.int32),
            pltpu.VMEM((2 * _GCH, M), jnp.float32),
            pltpu.VMEM((max(tail, 8),), jnp.int32),
            pltpu.VMEM((max(tail, 8), M), jnp.float32),
            pltpu.VMEM_SHARED((N, M), jnp.float32),
            pltpu.SemaphoreType.DMA,
        ],
    )
    return kfn(vals, idx, zeros_nm)


def _embed_body(x_ref, w_ref, b_ref, o_ref):
    o_ref[...] = (
        jnp.dot(x_ref[...], w_ref[...], preferred_element_type=jnp.float32, precision=lax.Precision.HIGHEST)
        + b_ref[...])


def _embed(x, w_t, b_row, block_rows):
    n, k = x.shape
    m = w_t.shape[1]
    grid = n // block_rows
    return pl.pallas_call(
        _embed_body,
        grid=(grid,),
        in_specs=[
            pl.BlockSpec((block_rows, k), lambda i: (i, 0)),
            pl.BlockSpec((k, m), lambda i: (0, 0)),
            pl.BlockSpec((1, m), lambda i: (0, 0)),
        ],
        out_specs=pl.BlockSpec((block_rows, m), lambda i: (i, 0)),
        out_shape=jax.ShapeDtypeStruct((n, m), jnp.float32),
    )(x, w_t, b_row)


def _edge_body(g1, g2, nbr, w1a, w1b, w1c, b1, w2, b2, w3, b3, ek_o, nbr_o):
    t = jnp.dot(g1[...], w1a[...], preferred_element_type=jnp.float32, precision=lax.Precision.HIGHEST)
    t += jnp.dot(g2[...], w1b[...], preferred_element_type=jnp.float32, precision=lax.Precision.HIGHEST)
    t += jnp.dot(nbr[...], w1c[...], preferred_element_type=jnp.float32, precision=lax.Precision.HIGHEST)
    h = _leaky(t + b1[...])
    h = _leaky(jnp.dot(h, w2[...], preferred_element_type=jnp.float32, precision=lax.Precision.HIGHEST)
               + b2[...])
    ek = jnp.dot(h, w3[...], preferred_element_type=jnp.float32, precision=lax.Precision.HIGHEST) + b3[...]
    ek_o[...] = ek
    nbr_o[...] = nbr[...] + ek


def _edge_mlp(g1, g2, nbr, w1a, w1b, w1c, b1, w2, b2, w3, b3, block_rows):
    grid = E // block_rows
    row = lambda i: (i, 0)
    fix = lambda i: (0, 0)
    return pl.pallas_call(
        _edge_body,
        grid=(grid,),
        in_specs=[
            pl.BlockSpec((block_rows, M), row),
            pl.BlockSpec((block_rows, M), row),
            pl.BlockSpec((block_rows, M), row),
            pl.BlockSpec((M, M), fix), pl.BlockSpec((M, M), fix),
            pl.BlockSpec((M, M), fix), pl.BlockSpec((1, M), fix),
            pl.BlockSpec((M, M), fix), pl.BlockSpec((1, M), fix),
            pl.BlockSpec((M, M), fix), pl.BlockSpec((1, M), fix),
        ],
        out_specs=[
            pl.BlockSpec((block_rows, M), row),
            pl.BlockSpec((block_rows, M), row),
        ],
        out_shape=[
            jax.ShapeDtypeStruct((E, M), jnp.float32),
            jax.ShapeDtypeStruct((E, M), jnp.float32),
        ],
    )(g1, g2, nbr, w1a, w1b, w1c, b1, w2, b2, w3, b3)


def _node_body(atom, p0, p1, nn, wa, wr, b1, w2, b2, w3, b3, g, bb, out):
    rho = (p0[...] + p1[...]) / nn[...]
    t = jnp.dot(atom[...], wa[...], preferred_element_type=jnp.float32, precision=lax.Precision.HIGHEST)
    t += jnp.dot(rho, wr[...], preferred_element_type=jnp.float32, precision=lax.Precision.HIGHEST)
    h = _leaky(t + b1[...])
    h = _leaky(jnp.dot(h, w2[...], preferred_element_type=jnp.float32, precision=lax.Precision.HIGHEST)
               + b2[...])
    vi = jnp.dot(h, w3[...], preferred_element_type=jnp.float32, precision=lax.Precision.HIGHEST) + b3[...]
    mu = jnp.mean(vi, axis=0, keepdims=True)
    var = jnp.mean((vi - mu) ** 2, axis=0, keepdims=True)
    vi = (vi - mu) / jnp.sqrt(var + 1e-5) * g[...] + bb[...]
    out[...] = atom[...] + vi


def _node_mlp(atom, p0, p1, nn_col, wa, wr, b1, w2, b2, w3, b3, g_row, b_row):
    return pl.pallas_call(
        _node_body,
        out_shape=jax.ShapeDtypeStruct((N, M), jnp.float32),
    )(atom, p0, p1, nn_col, wa, wr, b1, w2, b2, w3, b3, g_row, b_row)


def _pool_body(vi, p0, p1, nn, cidx, ua, ue, ub1, uw2, ub2,
               fcw, fcb, fc1w, fc1b, ow, ob, out,
               gfa, gfb, cnt):
    i = pl.program_id(0)
    nblk = pl.num_programs(0)
    rows = vi.shape[0]

    @pl.when(i == 0)
    def _zero():
        gfa[...] = jnp.zeros_like(gfa)
        gfb[...] = jnp.zeros_like(gfb)
        cnt[...] = jnp.zeros_like(cnt)

    eks = (p0[...] + p1[...]) / nn[...]
    iota = lax.broadcasted_iota(jnp.int32, (rows, NCRYS), 1)
    onehot = (iota == cidx[...]).astype(jnp.float32)
    dn = (((0,), (0,)), ((), ()))
    gfa[...] += lax.dot_general(onehot, vi[...], dn,
                                preferred_element_type=jnp.float32, precision=lax.Precision.HIGHEST)
    gfb[...] += lax.dot_general(onehot, eks, dn,
                                preferred_element_type=jnp.float32, precision=lax.Precision.HIGHEST)
    cnt[...] += lax.dot_general(onehot, jnp.ones((rows, M), jnp.float32), dn,
                                preferred_element_type=jnp.float32, precision=lax.Precision.HIGHEST)

    @pl.when(i == nblk - 1)
    def _head():
        pa = gfa[...] / cnt[...]
        pb = gfb[...] / cnt[...]
        z = jnp.dot(pa, ua[...], preferred_element_type=jnp.float32, precision=lax.Precision.HIGHEST)
        z += jnp.dot(pb, ue[...], preferred_element_type=jnp.float32, precision=lax.Precision.HIGHEST)
        z = _leaky(z + ub1[...])
        z = jnp.tanh(jnp.dot(z, uw2[...], preferred_element_type=jnp.float32, precision=lax.Precision.HIGHEST)
                     + ub2[...])
        c = _leaky(jnp.dot(z, fcw[...], preferred_element_type=jnp.float32, precision=lax.Precision.HIGHEST)
                   + fcb[...])
        c = _leaky(jnp.dot(c, fc1w[...], preferred_element_type=jnp.float32, precision=lax.Precision.HIGHEST)
                   + fc1b[...])
        out[...] = (jnp.dot(c, ow[...], preferred_element_type=jnp.float32, precision=lax.Precision.HIGHEST)
                    + ob[...])


def _pool_head(vi, p0, p1, nn_col, cidx_col, ua, ue, ub1, uw2, ub2,
               fcw, fcb, fc1w, fc1b, ow, ob, block_rows):
    grid = N // block_rows
    row = lambda i: (i, 0)
    fix = lambda i: (0, 0)
    return pl.pallas_call(
        _pool_body,
        grid=(grid,),
        in_specs=[
            pl.BlockSpec((block_rows, M), row),
            pl.BlockSpec((block_rows, M), row),
            pl.BlockSpec((block_rows, M), row),
            pl.BlockSpec((block_rows, 1), row),
            pl.BlockSpec((block_rows, 1), row),
            pl.BlockSpec((M, M), fix), pl.BlockSpec((M, M), fix),
            pl.BlockSpec((1, M), fix),
            pl.BlockSpec((M, M), fix), pl.BlockSpec((1, M), fix),
            pl.BlockSpec((M, M), fix), pl.BlockSpec((1, M), fix),
            pl.BlockSpec((M, M), fix), pl.BlockSpec((1, M), fix),
            pl.BlockSpec((M, 1), fix), pl.BlockSpec((1, 1), fix),
        ],
        out_specs=pl.BlockSpec((NCRYS, 1), fix),
        out_shape=jax.ShapeDtypeStruct((NCRYS, 1), jnp.float32),
        scratch_shapes=[
            pltpu.VMEM((NCRYS, M), jnp.float32),
            pltpu.VMEM((NCRYS, M), jnp.float32),
            pltpu.VMEM((NCRYS, M), jnp.float32),
        ],
    )(vi, p0, p1, nn_col, cidx_col, ua, ue, ub1, uw2, ub2,
      fcw, fcb, fc1w, fc1b, ow, ob)


def kernel(atom_fea, nbr_fea, nbr_fea_idx1, nbr_fea_idx2, num_nbrs,
           crystal_atom_idx, params):
    p = params
    rowb = lambda b: b.reshape(1, -1)

    idx_all = jnp.concatenate([nbr_fea_idx1, nbr_fea_idx2])
    stride = _NW * _GCH
    ep = ((2 * E + stride - 1) // stride) * stride
    idx_3d = jnp.pad(idx_all, (0, ep - 2 * E)).reshape(_NW, -1, _GCH)

    zeros_nm = jnp.zeros((N, M), jnp.float32)
    nn_col = num_nbrs.reshape(N, 1)
    cidx_col = crystal_atom_idx.reshape(N, 1)

    atom = _embed(atom_fea, p["node_W"].T, rowb(p["node_b"]), 2000)
    nbr = _embed(nbr_fea, p["edge_W"].T, rowb(p["edge_b"]), 2000)

    eks_parts = None
    nconv = len(p["convs"])
    for li, c in enumerate(p["convs"]):
        gath = _sc_gather(atom, idx_3d)
        e_w1t = c["eW1"].T
        ek, nbr = _edge_mlp(
            gath[:E], gath[E:2 * E], nbr,
            e_w1t[:M], e_w1t[M:2 * M], e_w1t[2 * M:], rowb(c["eb1"]),
            c["eW2"].T, rowb(c["eb2"]), c["eW3"].T, rowb(c["eb3"]), 2000)
        rho_parts = _sc_scatter(ek, nbr_fea_idx1, zeros_nm)
        v_w1t = c["vW1"].T
        atom = _node_mlp(
            atom, rho_parts[:N], rho_parts[N:], nn_col,
            v_w1t[:M], v_w1t[M:], rowb(c["vb1"]),
            c["vW2"].T, rowb(c["vb2"]), c["vW3"].T, rowb(c["vb3"]),
            rowb(c["bn_g"]), rowb(c["bn_b"]))
        if li == nconv - 1:
            eks_parts = _sc_scatter(nbr, nbr_fea_idx1, zeros_nm)

    u_w1t = p["uW1"].T
    return _pool_head(
        atom, eks_parts[:N], eks_parts[N:], nn_col, cidx_col,
        u_w1t[:M], u_w1t[M:], rowb(p["ub1"]),
        p["uW2"].T, rowb(p["ub2"]),
        p["fcW"].T, rowb(p["fcb"]),
        p["fc1W"].T, rowb(p["fc1b"]),
        p["outW"].T, rowb(p["outb"]), 2000)


# speedup vs baseline: 1.2810x; 1.0157x over previous
import functools

import jax
import jax.numpy as jnp
from jax import lax
from jax.experimental import pallas as pl
from jax.experimental.pallas import tpu as pltpu
from jax.experimental.pallas import tpu_sc as plsc

N = 10000
E = 320000
M = 128
NCRYS = 1024

_NC = 2
_NS = 16
_NW = _NC * _NS
_GCH = 128


def _leaky(x):
    return jnp.where(x >= 0, x, 0.2 * x)


def _sc_gather_body(table_hbm, idx_hbm, out_hbm, idx_v, rows_v, sem_g, sem_w):
    wid = lax.axis_index("s") * _NC + lax.axis_index("c")
    nch = idx_hbm.shape[1]
    base = wid * (nch * _GCH)

    pltpu.sync_copy(idx_hbm.at[wid], idx_v)

    def drain_w():
        pltpu.make_async_copy(
            rows_v.at[pl.ds(0, _GCH)],
            out_hbm.at[pl.ds(0, _GCH)], sem_w).wait()

    def fire_g(c):
        pltpu.async_copy(
            table_hbm.at[idx_v.at[c]],
            rows_v.at[pl.ds((c % 3) * _GCH, _GCH)], sem_g)

    fire_g(0)

    def body(c, _):
        @pl.when(c >= 2)
        def _wait_wb():
            drain_w()

        @pl.when(c + 1 < nch)
        def _next():
            fire_g(c + 1)

        pltpu.make_async_copy(
            out_hbm.at[pl.ds(0, _GCH)],
            rows_v.at[pl.ds(0, _GCH)], sem_g).wait()
        pltpu.async_copy(
            rows_v.at[pl.ds((c % 3) * _GCH, _GCH)],
            out_hbm.at[pl.ds(base + c * _GCH, _GCH)], sem_w)
        return 0

    lax.fori_loop(0, nch, body, 0)
    drain_w()
    drain_w()


def _sc_gather(table, idx_3d):
    nch = idx_3d.shape[1]
    ep = _NW * nch * _GCH
    kfn = pl.kernel(
        _sc_gather_body,
        out_type=jax.ShapeDtypeStruct((ep, M), jnp.float32),
        mesh=plsc.VectorSubcoreMesh(core_axis_name="c", subcore_axis_name="s"),
        scratch_types=[
            pltpu.VMEM((nch, _GCH), jnp.int32),
            pltpu.VMEM((3 * _GCH, M), jnp.float32),
            pltpu.SemaphoreType.DMA,
            pltpu.SemaphoreType.DMA,
        ],
    )
    return kfn(table, idx_3d)


def _sc_scatter_body(vals_hbm, idx_hbm, zeros_hbm, out_hbm,
                     idx_v, rows_v, idx_t, rows_t, accum, sem_l):
    cid = lax.axis_index("c")
    sid = lax.axis_index("s")
    wid = sid * _NC + cid
    per_w = vals_hbm.shape[0] // _NW
    nfull = per_w // _GCH
    tail = per_w - nfull * _GCH
    base = wid * per_w

    zstripe = (accum.shape[0] // _NS) // 8 * 8
    zlast = accum.shape[0] - zstripe * (_NS - 1)

    @pl.when(sid < _NS - 1)
    def _zero_main():
        pltpu.sync_copy(zeros_hbm.at[pl.ds(sid * zstripe, zstripe)],
                        accum.at[pl.ds(sid * zstripe, zstripe)])

    @pl.when(sid == _NS - 1)
    def _zero_last():
        pltpu.sync_copy(zeros_hbm.at[pl.ds(zstripe * (_NS - 1), zlast)],
                        accum.at[pl.ds(zstripe * (_NS - 1), zlast)])

    plsc.subcore_barrier()

    def fire(c):
        off = base + c * _GCH
        b = c % 2
        pltpu.async_copy(idx_hbm.at[pl.ds(off, _GCH)], idx_v.at[b], sem_l)
        pltpu.async_copy(vals_hbm.at[pl.ds(off, _GCH)],
                         rows_v.at[pl.ds(b * _GCH, _GCH)], sem_l)

    def drain_l():
        pltpu.make_async_copy(idx_hbm.at[pl.ds(0, _GCH)],
                              idx_v.at[0], sem_l).wait()
        pltpu.make_async_copy(vals_hbm.at[pl.ds(0, _GCH)],
                              rows_v.at[pl.ds(0, _GCH)], sem_l).wait()

    fire(0)

    def chunk(c, _):
        b = c % 2

        @pl.when(c + 1 < nfull)
        def _next():
            fire(c + 1)

        drain_l()
        pltpu.sync_copy(rows_v.at[pl.ds(b * _GCH, _GCH)],
                        accum.at[idx_v.at[b]], add=True)
        return 0

    lax.fori_loop(0, nfull, chunk, 0)

    if tail:
        toff = base + nfull * _GCH
        pltpu.sync_copy(idx_hbm.at[pl.ds(toff, tail)], idx_t)
        pltpu.sync_copy(vals_hbm.at[pl.ds(toff, tail)], rows_t)
        pltpu.sync_copy(rows_t, accum.at[idx_t], add=True)

    plsc.subcore_barrier()

    stripe = (accum.shape[0] // _NS) // 8 * 8
    last = accum.shape[0] - stripe * (_NS - 1)

    @pl.when(sid < _NS - 1)
    def _dump_main():
        pltpu.sync_copy(
            accum.at[pl.ds(sid * stripe, stripe)],
            out_hbm.at[pl.ds(cid * accum.shape[0] + sid * stripe, stripe)])

    @pl.when(sid == _NS - 1)
    def _dump_last():
        pltpu.sync_copy(
            accum.at[pl.ds(stripe * (_NS - 1), last)],
            out_hbm.at[pl.ds(cid * accum.shape[0] + stripe * (_NS - 1), last)])


def _sc_scatter(vals, idx, zeros_nm):
    per_w = vals.shape[0] // _NW
    tail = per_w - (per_w // _GCH) * _GCH
    kfn = pl.kernel(
        _sc_scatter_body,
        out_type=jax.ShapeDtypeStruct((2 * N, M), jnp.float32),
        mesh=plsc.VectorSubcoreMesh(core_axis_name="c", subcore_axis_name="s"),
        scratch_types=[
            pltpu.VMEM((2, _GCH), jnp.int32),
            pltpu.VMEM((2 * _GCH, M), jnp.float32),
            pltpu.VMEM((max(tail, 8),), jnp.int32),
            pltpu.VMEM((max(tail, 8), M), jnp.float32),
            pltpu.VMEM_SHARED((N, M), jnp.float32),
            pltpu.SemaphoreType.DMA,
        ],
    )
    return kfn(vals, idx, zeros_nm)


def _embed_body(x_ref, w_ref, b_ref, o_ref):
    o_ref[...] = (
        jnp.dot(x_ref[...], w_ref[...], preferred_element_type=jnp.float32, precision=lax.Precision.HIGHEST)
        + b_ref[...])


def _embed(x, w_t, b_row, block_rows):
    n, k = x.shape
    m = w_t.shape[1]
    grid = n // block_rows
    return pl.pallas_call(
        _embed_body,
        grid=(grid,),
        in_specs=[
            pl.BlockSpec((block_rows, k), lambda i: (i, 0)),
            pl.BlockSpec((k, m), lambda i: (0, 0)),
            pl.BlockSpec((1, m), lambda i: (0, 0)),
        ],
        out_specs=pl.BlockSpec((block_rows, m), lambda i: (i, 0)),
        out_shape=jax.ShapeDtypeStruct((n, m), jnp.float32),
    )(x, w_t, b_row)


def _edge_body(g1, g2, nbr_in, w1a, w1b, w1c, b1, w2, b2, w3, b3, ek_o, nbr_o,
               embed_w=None, embed_b=None):
    if embed_w is None:
        nbr = nbr_in[...]
    else:
        nbr = (jnp.dot(nbr_in[...], embed_w[...],
                       preferred_element_type=jnp.float32,
                       precision=lax.Precision.HIGHEST) + embed_b[...])
    t = jnp.dot(g1[...], w1a[...], preferred_element_type=jnp.float32, precision=lax.Precision.HIGHEST)
    t += jnp.dot(g2[...], w1b[...], preferred_element_type=jnp.float32, precision=lax.Precision.HIGHEST)
    t += jnp.dot(nbr, w1c[...], preferred_element_type=jnp.float32, precision=lax.Precision.HIGHEST)
    h = _leaky(t + b1[...])
    h = _leaky(jnp.dot(h, w2[...], preferred_element_type=jnp.float32, precision=lax.Precision.HIGHEST)
               + b2[...])
    ek = jnp.dot(h, w3[...], preferred_element_type=jnp.float32, precision=lax.Precision.HIGHEST) + b3[...]
    ek_o[...] = ek
    nbr_o[...] = nbr + ek


def _edge_mlp(g1, g2, nbr, w1a, w1b, w1c, b1, w2, b2, w3, b3, block_rows,
              embed=None):
    grid = E // block_rows
    row = lambda i: (i, 0)
    fix = lambda i: (0, 0)
    k = nbr.shape[1]
    body = _edge_body
    extra_specs = []
    extra_args = []
    if embed is not None:
        ew, eb = embed
        body = functools.partial(_edge_body)
        extra_specs = [pl.BlockSpec((k, M), fix), pl.BlockSpec((1, M), fix)]
        extra_args = [ew, eb]

        def body(g1, g2, nbr_in, w1a, w1b, w1c, b1, w2, b2, w3, b3,
                 ew_r, eb_r, ek_o, nbr_o):
            return _edge_body(g1, g2, nbr_in, w1a, w1b, w1c, b1, w2, b2,
                              w3, b3, ek_o, nbr_o, embed_w=ew_r, embed_b=eb_r)

    return pl.pallas_call(
        body,
        grid=(grid,),
        in_specs=[
            pl.BlockSpec((block_rows, M), row),
            pl.BlockSpec((block_rows, M), row),
            pl.BlockSpec((block_rows, k), row),
            pl.BlockSpec((M, M), fix), pl.BlockSpec((M, M), fix),
            pl.BlockSpec((M, M), fix), pl.BlockSpec((1, M), fix),
            pl.BlockSpec((M, M), fix), pl.BlockSpec((1, M), fix),
            pl.BlockSpec((M, M), fix), pl.BlockSpec((1, M), fix),
            *extra_specs,
        ],
        out_specs=[
            pl.BlockSpec((block_rows, M), row),
            pl.BlockSpec((block_rows, M), row),
        ],
        out_shape=[
            jax.ShapeDtypeStruct((E, M), jnp.float32),
            jax.ShapeDtypeStruct((E, M), jnp.float32),
        ],
    )(g1, g2, nbr, w1a, w1b, w1c, b1, w2, b2, w3, b3, *extra_args)


def _node_body(atom, p0, p1, nn, wa, wr, b1, w2, b2, w3, b3, g, bb, out):
    rho = (p0[...] + p1[...]) / nn[...]
    t = jnp.dot(atom[...], wa[...], preferred_element_type=jnp.float32, precision=lax.Precision.HIGHEST)
    t += jnp.dot(rho, wr[...], preferred_element_type=jnp.float32, precision=lax.Precision.HIGHEST)
    h = _leaky(t + b1[...])
    h = _leaky(jnp.dot(h, w2[...], preferred_element_type=jnp.float32, precision=lax.Precision.HIGHEST)
               + b2[...])
    vi = jnp.dot(h, w3[...], preferred_element_type=jnp.float32, precision=lax.Precision.HIGHEST) + b3[...]
    mu = jnp.mean(vi, axis=0, keepdims=True)
    var = jnp.mean((vi - mu) ** 2, axis=0, keepdims=True)
    vi = (vi - mu) / jnp.sqrt(var + 1e-5) * g[...] + bb[...]
    out[...] = atom[...] + vi


def _node_mlp(atom, p0, p1, nn_col, wa, wr, b1, w2, b2, w3, b3, g_row, b_row):
    return pl.pallas_call(
        _node_body,
        out_shape=jax.ShapeDtypeStruct((N, M), jnp.float32),
    )(atom, p0, p1, nn_col, wa, wr, b1, w2, b2, w3, b3, g_row, b_row)


def _pool_body(vi, p0, p1, nn, cidx, ua, ue, ub1, uw2, ub2,
               fcw, fcb, fc1w, fc1b, ow, ob, out,
               gfa, gfb, cnt):
    i = pl.program_id(0)
    nblk = pl.num_programs(0)
    rows = vi.shape[0]

    @pl.when(i == 0)
    def _zero():
        gfa[...] = jnp.zeros_like(gfa)
        gfb[...] = jnp.zeros_like(gfb)
        cnt[...] = jnp.zeros_like(cnt)

    eks = (p0[...] + p1[...]) / nn[...]
    iota = lax.broadcasted_iota(jnp.int32, (rows, NCRYS), 1)
    onehot = (iota == cidx[...]).astype(jnp.float32)
    dn = (((0,), (0,)), ((), ()))
    gfa[...] += lax.dot_general(onehot, vi[...], dn,
                                preferred_element_type=jnp.float32, precision=lax.Precision.HIGHEST)
    gfb[...] += lax.dot_general(onehot, eks, dn,
                                preferred_element_type=jnp.float32, precision=lax.Precision.HIGHEST)
    cnt[...] += lax.dot_general(onehot, jnp.ones((rows, M), jnp.float32), dn,
                                preferred_element_type=jnp.float32, precision=lax.Precision.HIGHEST)

    @pl.when(i == nblk - 1)
    def _head():
        pa = gfa[...] / cnt[...]
        pb = gfb[...] / cnt[...]
        z = jnp.dot(pa, ua[...], preferred_element_type=jnp.float32, precision=lax.Precision.HIGHEST)
        z += jnp.dot(pb, ue[...], preferred_element_type=jnp.float32, precision=lax.Precision.HIGHEST)
        z = _leaky(z + ub1[...])
        z = jnp.tanh(jnp.dot(z, uw2[...], preferred_element_type=jnp.float32, precision=lax.Precision.HIGHEST)
                     + ub2[...])
        c = _leaky(jnp.dot(z, fcw[...], preferred_element_type=jnp.float32, precision=lax.Precision.HIGHEST)
                   + fcb[...])
        c = _leaky(jnp.dot(c, fc1w[...], preferred_element_type=jnp.float32, precision=lax.Precision.HIGHEST)
                   + fc1b[...])
        out[...] = (jnp.dot(c, ow[...], preferred_element_type=jnp.float32, precision=lax.Precision.HIGHEST)
                    + ob[...])


def _pool_head(vi, p0, p1, nn_col, cidx_col, ua, ue, ub1, uw2, ub2,
               fcw, fcb, fc1w, fc1b, ow, ob, block_rows):
    grid = N // block_rows
    row = lambda i: (i, 0)
    fix = lambda i: (0, 0)
    return pl.pallas_call(
        _pool_body,
        grid=(grid,),
        in_specs=[
            pl.BlockSpec((block_rows, M), row),
            pl.BlockSpec((block_rows, M), row),
            pl.BlockSpec((block_rows, M), row),
            pl.BlockSpec((block_rows, 1), row),
            pl.BlockSpec((block_rows, 1), row),
            pl.BlockSpec((M, M), fix), pl.BlockSpec((M, M), fix),
            pl.BlockSpec((1, M), fix),
            pl.BlockSpec((M, M), fix), pl.BlockSpec((1, M), fix),
            pl.BlockSpec((M, M), fix), pl.BlockSpec((1, M), fix),
            pl.BlockSpec((M, M), fix), pl.BlockSpec((1, M), fix),
            pl.BlockSpec((M, 1), fix), pl.BlockSpec((1, 1), fix),
        ],
        out_specs=pl.BlockSpec((NCRYS, 1), fix),
        out_shape=jax.ShapeDtypeStruct((NCRYS, 1), jnp.float32),
        scratch_shapes=[
            pltpu.VMEM((NCRYS, M), jnp.float32),
            pltpu.VMEM((NCRYS, M), jnp.float32),
            pltpu.VMEM((NCRYS, M), jnp.float32),
        ],
    )(vi, p0, p1, nn_col, cidx_col, ua, ue, ub1, uw2, ub2,
      fcw, fcb, fc1w, fc1b, ow, ob)


def kernel(atom_fea, nbr_fea, nbr_fea_idx1, nbr_fea_idx2, num_nbrs,
           crystal_atom_idx, params):
    p = params
    rowb = lambda b: b.reshape(1, -1)

    idx_all = jnp.concatenate([nbr_fea_idx1, nbr_fea_idx2])
    stride = _NW * _GCH
    ep = ((2 * E + stride - 1) // stride) * stride
    idx_3d = jnp.pad(idx_all, (0, ep - 2 * E)).reshape(_NW, -1, _GCH)

    zeros_nm = jnp.zeros((N, M), jnp.float32)
    nn_col = num_nbrs.reshape(N, 1)
    cidx_col = crystal_atom_idx.reshape(N, 1)

    atom = _embed(atom_fea, p["node_W"].T, rowb(p["node_b"]), 2000)
    nbr = nbr_fea

    eks_parts = None
    nconv = len(p["convs"])
    for li, c in enumerate(p["convs"]):
        gath = _sc_gather(atom, idx_3d)
        e_w1t = c["eW1"].T
        embed = (p["edge_W"].T, rowb(p["edge_b"])) if li == 0 else None
        ek, nbr = _edge_mlp(
            gath[:E], gath[E:2 * E], nbr,
            e_w1t[:M], e_w1t[M:2 * M], e_w1t[2 * M:], rowb(c["eb1"]),
            c["eW2"].T, rowb(c["eb2"]), c["eW3"].T, rowb(c["eb3"]), 4000,
            embed=embed)
        rho_parts = _sc_scatter(ek, nbr_fea_idx1, zeros_nm)
        v_w1t = c["vW1"].T
        atom = _node_mlp(
            atom, rho_parts[:N], rho_parts[N:], nn_col,
            v_w1t[:M], v_w1t[M:], rowb(c["vb1"]),
            c["vW2"].T, rowb(c["vb2"]), c["vW3"].T, rowb(c["vb3"]),
            rowb(c["bn_g"]), rowb(c["bn_b"]))
        if li == nconv - 1:
            eks_parts = _sc_scatter(nbr, nbr_fea_idx1, zeros_nm)

    u_w1t = p["uW1"].T
    return _pool_head(
        atom, eks_parts[:N], eks_parts[N:], nn_col, cidx_col,
        u_w1t[:M], u_w1t[M:], rowb(p["ub1"]),
        p["uW2"].T, rowb(p["ub2"]),
        p["fcW"].T, rowb(p["fcb"]),
        p["fc1W"].T, rowb(p["fc1b"]),
        p["outW"].T, rowb(p["outb"]), 2000)


# speedup vs baseline: 2.3658x; 1.8469x over previous
import functools

import jax
import jax.numpy as jnp
from jax import lax
from jax.experimental import pallas as pl
from jax.experimental.pallas import tpu as pltpu
from jax.experimental.pallas import tpu_sc as plsc

N = 10000
E = 320000
M = 128
NCRYS = 1024

_NC = 2
_NS = 16
_NW = _NC * _NS
_GCH = 128


def _leaky(x):
    return jnp.where(x >= 0, x, 0.2 * x)


def _sc_gather_body(table_hbm, idx_hbm, out_hbm, idx_v, rows_v, sem_g, sem_w):
    wid = lax.axis_index("s") * _NC + lax.axis_index("c")
    nch = idx_hbm.shape[1]
    base = wid * (nch * _GCH)

    pltpu.sync_copy(idx_hbm.at[wid], idx_v)

    def drain_w():
        pltpu.make_async_copy(
            rows_v.at[pl.ds(0, _GCH)],
            out_hbm.at[pl.ds(0, _GCH)], sem_w).wait()

    def fire_g(c):
        pltpu.async_copy(
            table_hbm.at[idx_v.at[c]],
            rows_v.at[pl.ds((c % 3) * _GCH, _GCH)], sem_g)

    fire_g(0)

    def body(c, _):
        @pl.when(c >= 2)
        def _wait_wb():
            drain_w()

        @pl.when(c + 1 < nch)
        def _next():
            fire_g(c + 1)

        pltpu.make_async_copy(
            out_hbm.at[pl.ds(0, _GCH)],
            rows_v.at[pl.ds(0, _GCH)], sem_g).wait()
        pltpu.async_copy(
            rows_v.at[pl.ds((c % 3) * _GCH, _GCH)],
            out_hbm.at[pl.ds(base + c * _GCH, _GCH)], sem_w)
        return 0

    lax.fori_loop(0, nch, body, 0)
    drain_w()
    drain_w()


def _sc_gather(table, idx_3d):
    nch = idx_3d.shape[1]
    ep = _NW * nch * _GCH
    kfn = pl.kernel(
        _sc_gather_body,
        out_type=jax.ShapeDtypeStruct((ep, M), jnp.float32),
        mesh=plsc.VectorSubcoreMesh(core_axis_name="c", subcore_axis_name="s"),
        scratch_types=[
            pltpu.VMEM((nch, _GCH), jnp.int32),
            pltpu.VMEM((3 * _GCH, M), jnp.float32),
            pltpu.SemaphoreType.DMA,
            pltpu.SemaphoreType.DMA,
        ],
    )
    return kfn(table, idx_3d)


def _sc_scatter_body(vals_hbm, idx_hbm, zeros_hbm, out_hbm,
                     idx_v, rows_v, idx_t, rows_t, accum, sem_l):
    cid = lax.axis_index("c")
    sid = lax.axis_index("s")
    wid = sid * _NC + cid
    per_w = vals_hbm.shape[0] // _NW
    nfull = per_w // _GCH
    tail = per_w - nfull * _GCH
    base = wid * per_w

    zstripe = (accum.shape[0] // _NS) // 8 * 8
    zlast = accum.shape[0] - zstripe * (_NS - 1)

    @pl.when(sid < _NS - 1)
    def _zero_main():
        pltpu.sync_copy(zeros_hbm.at[pl.ds(sid * zstripe, zstripe)],
                        accum.at[pl.ds(sid * zstripe, zstripe)])

    @pl.when(sid == _NS - 1)
    def _zero_last():
        pltpu.sync_copy(zeros_hbm.at[pl.ds(zstripe * (_NS - 1), zlast)],
                        accum.at[pl.ds(zstripe * (_NS - 1), zlast)])

    plsc.subcore_barrier()

    def fire(c):
        off = base + c * _GCH
        b = c % 2
        pltpu.async_copy(idx_hbm.at[pl.ds(off, _GCH)], idx_v.at[b], sem_l)
        pltpu.async_copy(vals_hbm.at[pl.ds(off, _GCH)],
                         rows_v.at[pl.ds(b * _GCH, _GCH)], sem_l)

    def drain_l():
        pltpu.make_async_copy(idx_hbm.at[pl.ds(0, _GCH)],
                              idx_v.at[0], sem_l).wait()
        pltpu.make_async_copy(vals_hbm.at[pl.ds(0, _GCH)],
                              rows_v.at[pl.ds(0, _GCH)], sem_l).wait()

    fire(0)

    def chunk(c, _):
        b = c % 2

        @pl.when(c + 1 < nfull)
        def _next():
            fire(c + 1)

        drain_l()
        pltpu.sync_copy(rows_v.at[pl.ds(b * _GCH, _GCH)],
                        accum.at[idx_v.at[b]], add=True)
        return 0

    lax.fori_loop(0, nfull, chunk, 0)

    if tail:
        toff = base + nfull * _GCH
        pltpu.sync_copy(idx_hbm.at[pl.ds(toff, tail)], idx_t)
        pltpu.sync_copy(vals_hbm.at[pl.ds(toff, tail)], rows_t)
        pltpu.sync_copy(rows_t, accum.at[idx_t], add=True)

    plsc.subcore_barrier()

    stripe = (accum.shape[0] // _NS) // 8 * 8
    last = accum.shape[0] - stripe * (_NS - 1)

    @pl.when(sid < _NS - 1)
    def _dump_main():
        pltpu.sync_copy(
            accum.at[pl.ds(sid * stripe, stripe)],
            out_hbm.at[pl.ds(cid * accum.shape[0] + sid * stripe, stripe)])

    @pl.when(sid == _NS - 1)
    def _dump_last():
        pltpu.sync_copy(
            accum.at[pl.ds(stripe * (_NS - 1), last)],
            out_hbm.at[pl.ds(cid * accum.shape[0] + stripe * (_NS - 1), last)])


def _sc_scatter(vals, idx, zeros_nm):
    per_w = vals.shape[0] // _NW
    tail = per_w - (per_w // _GCH) * _GCH
    kfn = pl.kernel(
        _sc_scatter_body,
        out_type=jax.ShapeDtypeStruct((2 * N, M), jnp.float32),
        mesh=plsc.VectorSubcoreMesh(core_axis_name="c", subcore_axis_name="s"),
        scratch_types=[
            pltpu.VMEM((2, _GCH), jnp.int32),
            pltpu.VMEM((2 * _GCH, M), jnp.float32),
            pltpu.VMEM((max(tail, 8),), jnp.int32),
            pltpu.VMEM((max(tail, 8), M), jnp.float32),
            pltpu.VMEM_SHARED((N, M), jnp.float32),
            pltpu.SemaphoreType.DMA,
        ],
    )
    return kfn(vals, idx, zeros_nm)


def _embed_body(x_ref, w_ref, b_ref, o_ref):
    o_ref[...] = (
        jnp.dot(x_ref[...], w_ref[...], preferred_element_type=jnp.float32, precision=lax.Precision.DEFAULT)
        + b_ref[...])


def _embed(x, w_t, b_row, block_rows):
    n, k = x.shape
    m = w_t.shape[1]
    grid = n // block_rows
    return pl.pallas_call(
        _embed_body,
        grid=(grid,),
        in_specs=[
            pl.BlockSpec((block_rows, k), lambda i: (i, 0)),
            pl.BlockSpec((k, m), lambda i: (0, 0)),
            pl.BlockSpec((1, m), lambda i: (0, 0)),
        ],
        out_specs=pl.BlockSpec((block_rows, m), lambda i: (i, 0)),
        out_shape=jax.ShapeDtypeStruct((n, m), jnp.float32),
    )(x, w_t, b_row)


def _edge_body(g1, g2, nbr_in, w1a, w1b, w1c, b1, w2, b2, w3, b3, ek_o, nbr_o,
               embed_w=None, embed_b=None):
    if embed_w is None:
        nbr = nbr_in[...]
    else:
        nbr = (jnp.dot(nbr_in[...], embed_w[...],
                       preferred_element_type=jnp.float32,
                       precision=lax.Precision.DEFAULT) + embed_b[...])
    t = jnp.dot(g1[...], w1a[...], preferred_element_type=jnp.float32, precision=lax.Precision.DEFAULT)
    t += jnp.dot(g2[...], w1b[...], preferred_element_type=jnp.float32, precision=lax.Precision.DEFAULT)
    t += jnp.dot(nbr, w1c[...], preferred_element_type=jnp.float32, precision=lax.Precision.DEFAULT)
    h = _leaky(t + b1[...])
    h = _leaky(jnp.dot(h, w2[...], preferred_element_type=jnp.float32, precision=lax.Precision.DEFAULT)
               + b2[...])
    ek = jnp.dot(h, w3[...], preferred_element_type=jnp.float32, precision=lax.Precision.DEFAULT) + b3[...]
    ek_o[...] = ek
    nbr_o[...] = nbr + ek


def _edge_mlp(g1, g2, nbr, w1a, w1b, w1c, b1, w2, b2, w3, b3, block_rows,
              embed=None):
    grid = E // block_rows
    row = lambda i: (i, 0)
    fix = lambda i: (0, 0)
    k = nbr.shape[1]
    body = _edge_body
    extra_specs = []
    extra_args = []
    if embed is not None:
        ew, eb = embed
        body = functools.partial(_edge_body)
        extra_specs = [pl.BlockSpec((k, M), fix), pl.BlockSpec((1, M), fix)]
        extra_args = [ew, eb]

        def body(g1, g2, nbr_in, w1a, w1b, w1c, b1, w2, b2, w3, b3,
                 ew_r, eb_r, ek_o, nbr_o):
            return _edge_body(g1, g2, nbr_in, w1a, w1b, w1c, b1, w2, b2,
                              w3, b3, ek_o, nbr_o, embed_w=ew_r, embed_b=eb_r)

    return pl.pallas_call(
        body,
        grid=(grid,),
        in_specs=[
            pl.BlockSpec((block_rows, M), row),
            pl.BlockSpec((block_rows, M), row),
            pl.BlockSpec((block_rows, k), row),
            pl.BlockSpec((M, M), fix), pl.BlockSpec((M, M), fix),
            pl.BlockSpec((M, M), fix), pl.BlockSpec((1, M), fix),
            pl.BlockSpec((M, M), fix), pl.BlockSpec((1, M), fix),
            pl.BlockSpec((M, M), fix), pl.BlockSpec((1, M), fix),
            *extra_specs,
        ],
        out_specs=[
            pl.BlockSpec((block_rows, M), row),
            pl.BlockSpec((block_rows, M), row),
        ],
        out_shape=[
            jax.ShapeDtypeStruct((E, M), jnp.float32),
            jax.ShapeDtypeStruct((E, M), jnp.float32),
        ],
    )(g1, g2, nbr, w1a, w1b, w1c, b1, w2, b2, w3, b3, *extra_args)


def _node_body(atom, p0, p1, nn, wa, wr, b1, w2, b2, w3, b3, g, bb, out):
    rho = (p0[...] + p1[...]) / nn[...]
    t = jnp.dot(atom[...], wa[...], preferred_element_type=jnp.float32, precision=lax.Precision.DEFAULT)
    t += jnp.dot(rho, wr[...], preferred_element_type=jnp.float32, precision=lax.Precision.DEFAULT)
    h = _leaky(t + b1[...])
    h = _leaky(jnp.dot(h, w2[...], preferred_element_type=jnp.float32, precision=lax.Precision.DEFAULT)
               + b2[...])
    vi = jnp.dot(h, w3[...], preferred_element_type=jnp.float32, precision=lax.Precision.DEFAULT) + b3[...]
    mu = jnp.mean(vi, axis=0, keepdims=True)
    var = jnp.mean((vi - mu) ** 2, axis=0, keepdims=True)
    vi = (vi - mu) / jnp.sqrt(var + 1e-5) * g[...] + bb[...]
    out[...] = atom[...] + vi


def _node_mlp(atom, p0, p1, nn_col, wa, wr, b1, w2, b2, w3, b3, g_row, b_row):
    return pl.pallas_call(
        _node_body,
        out_shape=jax.ShapeDtypeStruct((N, M), jnp.float32),
    )(atom, p0, p1, nn_col, wa, wr, b1, w2, b2, w3, b3, g_row, b_row)


def _pool_body(vi, p0, p1, nn, cidx, ua, ue, ub1, uw2, ub2, out,
               gfa, gfb, cnt):
    i = pl.program_id(0)
    nblk = pl.num_programs(0)
    rows = vi.shape[0]

    @pl.when(i == 0)
    def _zero():
        gfa[...] = jnp.zeros_like(gfa)
        gfb[...] = jnp.zeros_like(gfb)
        cnt[...] = jnp.zeros_like(cnt)

    eks = (p0[...] + p1[...]) / nn[...]
    iota = lax.broadcasted_iota(jnp.int32, (rows, NCRYS), 1)
    onehot = (iota == cidx[...]).astype(jnp.float32)
    dn = (((0,), (0,)), ((), ()))
    gfa[...] += lax.dot_general(onehot, vi[...], dn,
                                preferred_element_type=jnp.float32, precision=lax.Precision.HIGHEST)
    gfb[...] += lax.dot_general(onehot, eks, dn,
                                preferred_element_type=jnp.float32, precision=lax.Precision.HIGHEST)
    cnt[...] += lax.dot_general(onehot, jnp.ones((rows, M), jnp.float32), dn,
                                preferred_element_type=jnp.float32, precision=lax.Precision.HIGHEST)

    @pl.when(i == nblk - 1)
    def _head():
        pa = gfa[...] / cnt[...]
        pb = gfb[...] / cnt[...]
        z = jnp.dot(pa, ua[...], preferred_element_type=jnp.float32, precision=lax.Precision.DEFAULT)
        z += jnp.dot(pb, ue[...], preferred_element_type=jnp.float32, precision=lax.Precision.DEFAULT)
        z = _leaky(z + ub1[...])
        out[...] = (jnp.dot(z, uw2[...], preferred_element_type=jnp.float32, precision=lax.Precision.DEFAULT)
                    + ub2[...])


def _head_body(z, fcw, fcb, fc1w, fc1b, ow, ob, out):
    c = _leaky(jnp.dot(z[...], fcw[...], preferred_element_type=jnp.float32, precision=lax.Precision.DEFAULT)
               + fcb[...])
    c = _leaky(jnp.dot(c, fc1w[...], preferred_element_type=jnp.float32, precision=lax.Precision.DEFAULT)
               + fc1b[...])
    out[...] = (jnp.dot(c, ow[...], preferred_element_type=jnp.float32, precision=lax.Precision.DEFAULT)
                + ob[...])


def _head(z, fcw, fcb, fc1w, fc1b, ow, ob):
    return pl.pallas_call(
        _head_body,
        out_shape=jax.ShapeDtypeStruct((NCRYS, 1), jnp.float32),
    )(z, fcw, fcb, fc1w, fc1b, ow, ob)


def _pool_head(vi, p0, p1, nn_col, cidx_col, ua, ue, ub1, uw2, ub2,
               block_rows):
    grid = N // block_rows
    row = lambda i: (i, 0)
    fix = lambda i: (0, 0)
    return pl.pallas_call(
        _pool_body,
        grid=(grid,),
        in_specs=[
            pl.BlockSpec((block_rows, M), row),
            pl.BlockSpec((block_rows, M), row),
            pl.BlockSpec((block_rows, M), row),
            pl.BlockSpec((block_rows, 1), row),
            pl.BlockSpec((block_rows, 1), row),
            pl.BlockSpec((M, M), fix), pl.BlockSpec((M, M), fix),
            pl.BlockSpec((1, M), fix),
            pl.BlockSpec((M, M), fix), pl.BlockSpec((1, M), fix),
        ],
        out_specs=pl.BlockSpec((NCRYS, M), fix),
        out_shape=jax.ShapeDtypeStruct((NCRYS, M), jnp.float32),
        scratch_shapes=[
            pltpu.VMEM((NCRYS, M), jnp.float32),
            pltpu.VMEM((NCRYS, M), jnp.float32),
            pltpu.VMEM((NCRYS, M), jnp.float32),
        ],
    )(vi, p0, p1, nn_col, cidx_col, ua, ue, ub1, uw2, ub2)


def kernel(atom_fea, nbr_fea, nbr_fea_idx1, nbr_fea_idx2, num_nbrs,
           crystal_atom_idx, params):
    p = params
    rowb = lambda b: b.reshape(1, -1)

    idx_all = jnp.concatenate([nbr_fea_idx1, nbr_fea_idx2])
    stride = _NW * _GCH
    ep = ((2 * E + stride - 1) // stride) * stride
    idx_3d = jnp.pad(idx_all, (0, ep - 2 * E)).reshape(_NW, -1, _GCH)

    zeros_nm = jnp.zeros((N, M), jnp.float32)
    nn_col = num_nbrs.reshape(N, 1)
    cidx_col = crystal_atom_idx.reshape(N, 1)

    atom = _embed(atom_fea, p["node_W"].T, rowb(p["node_b"]), 2000)
    nbr = nbr_fea

    eks_parts = None
    nconv = len(p["convs"])
    for li, c in enumerate(p["convs"]):
        gath = _sc_gather(atom, idx_3d)
        e_w1t = c["eW1"].T
        embed = (p["edge_W"].T, rowb(p["edge_b"])) if li == 0 else None
        ek, nbr = _edge_mlp(
            gath[:E], gath[E:2 * E], nbr,
            e_w1t[:M], e_w1t[M:2 * M], e_w1t[2 * M:], rowb(c["eb1"]),
            c["eW2"].T, rowb(c["eb2"]), c["eW3"].T, rowb(c["eb3"]), 4000,
            embed=embed)
        rho_parts = _sc_scatter(ek, nbr_fea_idx1, zeros_nm)
        v_w1t = c["vW1"].T
        atom = _node_mlp(
            atom, rho_parts[:N], rho_parts[N:], nn_col,
            v_w1t[:M], v_w1t[M:], rowb(c["vb1"]),
            c["vW2"].T, rowb(c["vb2"]), c["vW3"].T, rowb(c["vb3"]),
            rowb(c["bn_g"]), rowb(c["bn_b"]))
        if li == nconv - 1:
            eks_parts = _sc_scatter(nbr, nbr_fea_idx1, zeros_nm)

    u_w1t = p["uW1"].T
    zpre = _pool_head(
        atom, eks_parts[:N], eks_parts[N:], nn_col, cidx_col,
        u_w1t[:M], u_w1t[M:], rowb(p["ub1"]),
        p["uW2"].T, rowb(p["ub2"]), 2000)
    z = jnp.tanh(zpre)
    return _head(z, p["fcW"].T, rowb(p["fcb"]),
                 p["fc1W"].T, rowb(p["fc1b"]),
                 p["outW"].T, rowb(p["outb"]))
